# Initial kernel scaffold; baseline (speedup 1.0000x reference)
import jax
import jax.numpy as jnp
from jax.experimental import pallas as pl


def kernel(boxes, scores):
    raise NotImplementedError("write your pallas kernel here")


# speedup vs baseline: 7.5885x; 7.5885x over previous
import jax
import jax.numpy as jnp
from jax import lax
from jax.experimental import pallas as pl

N = 20000
NPAD = 20480
ROWS = NPAD // 128
K1 = 400
K2 = 200
CPAD = 512
FPAD = 256
SENT = -(2 ** 31)
IBIG = 2 ** 31 - 1
NEG_INF = -1e10
SCORE_THRESH = 0.01
NMS_THRESH = 0.45


def _body(sc_ref, x1_ref, y1_ref, x2_ref, y2_ref,
          ox1_ref, oy1_ref, ox2_ref, oy2_ref, osc_ref):
    scores = sc_ref[...]
    X1 = x1_ref[...]
    Y1 = y1_ref[...]
    X2 = x2_ref[...]
    Y2 = y2_ref[...]

    key0 = lax.bitcast_convert_type(scores, jnp.int32)
    key0 = jnp.where(scores > SCORE_THRESH, key0, SENT)
    fiota = (lax.broadcasted_iota(jnp.int32, (ROWS, 128), 0) * 128
             + lax.broadcasted_iota(jnp.int32, (ROWS, 128), 1))
    pos512 = (lax.broadcasted_iota(jnp.int32, (4, 128), 0) * 128
              + lax.broadcasted_iota(jnp.int32, (4, 128), 1))
    pos256 = (lax.broadcasted_iota(jnp.int32, (2, 128), 0) * 128
              + lax.broadcasted_iota(jnp.int32, (2, 128), 1))

    zc = jnp.zeros((4, 128), jnp.float32)
    ckey0 = jnp.full((4, 128), SENT, jnp.int32)

    def sel_body(k, carry):
        key, ckey, cx1, cy1, cx2, cy2 = carry
        m = jnp.max(key)
        pos = jnp.min(jnp.where(key == m, fiota, IBIG))
        hit = fiota == pos
        key = jnp.where(hit, SENT, key)
        bx1 = jnp.sum(jnp.where(hit, X1, 0.0))
        by1 = jnp.sum(jnp.where(hit, Y1, 0.0))
        bx2 = jnp.sum(jnp.where(hit, X2, 0.0))
        by2 = jnp.sum(jnp.where(hit, Y2, 0.0))
        mask = pos512 == k
        ckey = jnp.where(mask, m, ckey)
        cx1 = jnp.where(mask, bx1, cx1)
        cy1 = jnp.where(mask, by1, cy1)
        cx2 = jnp.where(mask, bx2, cx2)
        cy2 = jnp.where(mask, by2, cy2)
        return key, ckey, cx1, cy1, cx2, cy2

    _, ckey, cx1, cy1, cx2, cy2 = lax.fori_loop(
        0, K1, sel_body, (key0, ckey0, zc, zc, zc, zc))

    a2 = (cx2 - cx1) * (cy2 - cy1)

    def nms_body(i, sup):
        hit = pos512 == i
        bx1 = jnp.sum(jnp.where(hit, cx1, 0.0))
        by1 = jnp.sum(jnp.where(hit, cy1, 0.0))
        bx2 = jnp.sum(jnp.where(hit, cx2, 0.0))
        by2 = jnp.sum(jnp.where(hit, cy2, 0.0))
        si = jnp.max(jnp.where(hit, sup, 0))
        xx1 = jnp.maximum(bx1, cx1)
        yy1 = jnp.maximum(by1, cy1)
        xx2 = jnp.minimum(bx2, cx2)
        yy2 = jnp.minimum(by2, cy2)
        inter = jnp.maximum(xx2 - xx1, 0.0) * jnp.maximum(yy2 - yy1, 0.0)
        a1 = (bx2 - bx1) * (by2 - by1)
        iou = inter / (a1 + a2 - inter + jnp.float32(1e-9))
        new = sup | (((iou >= NMS_THRESH) & (pos512 > i)).astype(jnp.int32))
        return jnp.where(si > 0, sup, new)

    sup = lax.fori_loop(0, K1, nms_body, jnp.zeros((4, 128), jnp.int32))

    keep0 = jnp.where(sup > 0, SENT, ckey)
    zf = jnp.zeros((2, 128), jnp.float32)

    def fin_body(k, carry):
        keep, fx1, fy1, fx2, fy2, fsc = carry
        m2 = jnp.max(keep)
        p2 = jnp.min(jnp.where(keep == m2, pos512, IBIG))
        hit = pos512 == p2
        keep = jnp.where(hit, SENT, keep)
        bx1 = jnp.sum(jnp.where(hit, cx1, 0.0))
        by1 = jnp.sum(jnp.where(hit, cy1, 0.0))
        bx2 = jnp.sum(jnp.where(hit, cx2, 0.0))
        by2 = jnp.sum(jnp.where(hit, cy2, 0.0))
        sc = jnp.where(m2 == SENT, NEG_INF,
                       lax.bitcast_convert_type(m2, jnp.float32))
        mask = pos256 == k
        fx1 = jnp.where(mask, bx1, fx1)
        fy1 = jnp.where(mask, by1, fy1)
        fx2 = jnp.where(mask, bx2, fx2)
        fy2 = jnp.where(mask, by2, fy2)
        fsc = jnp.where(mask, sc, fsc)
        return keep, fx1, fy1, fx2, fy2, fsc

    _, fx1, fy1, fx2, fy2, fsc = lax.fori_loop(
        0, K2, fin_body, (keep0, zf, zf, zf, zf, zf))

    ox1_ref[...] = fx1
    oy1_ref[...] = fy1
    ox2_ref[...] = fx2
    oy2_ref[...] = fy2
    osc_ref[...] = fsc


def kernel(boxes, scores):
    scp = jnp.pad(scores, (0, NPAD - N)).reshape(ROWS, 128)
    bp = jnp.pad(boxes, ((0, NPAD - N), (0, 0)))
    comps = [bp[:, i].reshape(ROWS, 128) for i in range(4)]
    outs = pl.pallas_call(
        _body,
        out_shape=[jax.ShapeDtypeStruct((2, 128), jnp.float32)] * 5,
    )(scp, *comps)
    cols = [o.reshape(-1)[:K2] for o in outs]
    return jnp.stack(cols, axis=1)
